# baseline (speedup 1.0000x reference)
import functools

import jax
import jax.numpy as jnp
from jax import lax
from jax.experimental import pallas as pl
from jax.experimental.pallas import tpu as pltpu
from jax.experimental.pallas import tpu_sc as plsc

NUM_CORES = 2
NUM_SUBCORES = 16
LANES = 16
GROUP_B = 4


def _make_sc_kernel(B, S, C):
    NW = NUM_CORES * NUM_SUBCORES
    rows_per_w = B // NW
    n_groups = rows_per_w // GROUP_B
    group_elems = GROUP_B * S
    n_vecs = group_elems // LANES
    mesh = plsc.VectorSubcoreMesh(core_axis_name="c", subcore_axis_name="s")

    @functools.partial(
        pl.kernel,
        mesh=mesh,
        compiler_params=pltpu.CompilerParams(use_tc_tiling_on_sc=False, needs_layout_passes=False),
        out_type=jax.ShapeDtypeStruct((B, C, S), jnp.float32),
        scratch_types=[
            pltpu.VMEM((rows_per_w * S,), jnp.int32),
            pltpu.VMEM((GROUP_B, C, S), jnp.float32),
            pltpu.SemaphoreType.DMA,
        ],
    )
    def k(t_hbm, z_hbm, out_hbm, t_v, slab, sem):
        wid = lax.axis_index("s") * NUM_CORES + lax.axis_index("c")
        row0 = wid * rows_per_w
        pltpu.sync_copy(t_hbm.at[pl.ds(wid * rows_per_w * S, rows_per_w * S)], t_v)
        pltpu.sync_copy(z_hbm, slab)

        def scat(g, val):
            base = g * group_elems
            lane = lax.iota(jnp.int32, 16)
            vals = jnp.full((LANES,), val, jnp.float32)
            for j in range(n_vecs):
                f = lane + (j * LANES)
                tv = t_v[pl.ds(base + j * LANES, LANES)]
                plsc.store_scatter(slab, [f // S, tv, f % S], vals)

        def body_fn(g, carry):
            @pl.when(g > 0)
            def _wait_and_clear():
                pltpu.make_async_copy(
                    slab, out_hbm.at[pl.ds(row0, GROUP_B)], sem
                ).wait()
                scat(g - 1, 0.0)

            scat(g, 1.0)
            pltpu.async_copy(
                slab, out_hbm.at[pl.ds(row0 + g * GROUP_B, GROUP_B)], sem
            )
            return carry

        lax.fori_loop(0, n_groups, body_fn, 0)
        pltpu.make_async_copy(slab, out_hbm.at[pl.ds(row0, GROUP_B)], sem).wait()

    return k


def kernel(t, ones):
    B, S = t.shape
    C = ones.shape[0]
    t1d = t.astype(jnp.int32).reshape(-1)
    zeros = jnp.zeros((GROUP_B, C, S), jnp.float32)
    return _make_sc_kernel(B, S, C)(t1d, zeros)


# speedup vs baseline: 46.3284x; 46.3284x over previous
import jax
import jax.numpy as jnp
from jax.experimental import pallas as pl

B_TILE = 1024


def _onehot_block(t_ref, out_ref):
    tb = t_ref[...]
    cls = jax.lax.broadcasted_iota(jnp.int32, out_ref.shape, 1)
    out_ref[...] = (tb == cls).astype(jnp.float32)


def kernel(t, ones):
    B, S = t.shape
    C = ones.shape[0]
    t3 = t.astype(jnp.int32).T.reshape(S, 1, B)
    out_t = pl.pallas_call(
        _onehot_block,
        grid=(S, B // B_TILE),
        in_specs=[pl.BlockSpec((1, 1, B_TILE), lambda s, j: (s, 0, j))],
        out_specs=pl.BlockSpec((1, C, B_TILE), lambda s, j: (s, 0, j)),
        out_shape=jax.ShapeDtypeStruct((S, C, B), jnp.float32),
    )(t3)
    return jnp.transpose(out_t, (2, 1, 0))
